# baseline (speedup 1.0000x reference)
import functools

import jax
import jax.numpy as jnp
from jax import lax
from jax.experimental import pallas as pl
from jax.experimental.pallas import tpu as pltpu
from jax.experimental.pallas import tpu_sc as plsc

N_NODES = 10000
N_EDGES = 320000
D_FEAT = 128

NC = 2
NS = 16
NW = NC * NS
E_PER_W = N_EDGES // NW
CHUNK = 80
N_CHUNKS = E_PER_W // CHUNK
NB = 2
N_PAD = 10112
RPT = N_PAD // NS


def _elu_body(x_ref, w_ref, o_ref):
    xw = x_ref[...] * w_ref[...]
    o_ref[...] = jnp.where(xw > 0, xw, jnp.exp(xw) - 1.0)


@jax.jit
def _elu(x, w):
    blk = 400
    return pl.pallas_call(
        _elu_body,
        grid=(N_NODES // blk,),
        in_specs=[pl.BlockSpec((blk, D_FEAT), lambda i: (i, 0)),
                  pl.BlockSpec((1, D_FEAT), lambda i: (0, 0))],
        out_specs=pl.BlockSpec((blk, D_FEAT), lambda i: (i, 0)),
        out_shape=jax.ShapeDtypeStruct((N_NODES, D_FEAT), jnp.float32),
    )(x, w)


def _sum_body(p_ref, o_ref):
    o_ref[...] = p_ref[0] + p_ref[1]


@jax.jit
def _sum_partials(p):
    blk = 632
    out = pl.pallas_call(
        _sum_body,
        grid=(N_PAD // blk,),
        in_specs=[pl.BlockSpec((NC, blk, D_FEAT), lambda i: (0, i, 0))],
        out_specs=pl.BlockSpec((blk, D_FEAT), lambda i: (i, 0)),
        out_shape=jax.ShapeDtypeStruct((N_PAD, D_FEAT), jnp.float32),
    )(p)
    return out[:N_NODES]


def _scatter_body(h_hbm, src_hbm, dst_hbm, zeros_hbm, out_hbm,
                  acc_sh, srcb, dstb, rows, sem0, sem1, psem):
    c = lax.axis_index("c")
    s = lax.axis_index("s")
    wid = s * NC + c
    sems = [sem0, sem1]

    acc_slice = acc_sh.at[pl.ds(s * RPT, RPT)]
    pltpu.async_copy(src_hbm.at[wid], srcb, psem)
    pltpu.async_copy(dst_hbm.at[wid], dstb, psem)
    pltpu.async_copy(zeros_hbm, acc_slice, psem)
    pltpu.make_async_copy(src_hbm.at[wid], srcb, psem).wait()
    pltpu.make_async_copy(dst_hbm.at[wid], dstb, psem).wait()
    pltpu.make_async_copy(zeros_hbm, acc_slice, psem).wait()
    plsc.subcore_barrier()

    def idx_slice(ref, i):
        return ref.at[pl.ds(pl.multiple_of(i * CHUNK, 8), CHUNK)]

    for b in range(NB):
        pltpu.async_copy(h_hbm.at[idx_slice(srcb, b)], rows.at[b], sems[b])

    def pair(j, carry):
        for b in range(NB):
            i = j * NB + b

            @pl.when(i < N_CHUNKS)
            def _():
                pltpu.make_async_copy(h_hbm.at[idx_slice(srcb, i)],
                                      rows.at[b], sems[b]).wait()
                pltpu.sync_copy(rows.at[b], acc_sh.at[idx_slice(dstb, i)],
                                add=True)
                nxt = i + NB

                @pl.when(nxt < N_CHUNKS)
                def _():
                    pltpu.async_copy(h_hbm.at[idx_slice(srcb, nxt)],
                                     rows.at[b], sems[b])

        return carry

    lax.fori_loop(0, (N_CHUNKS + NB - 1) // NB, pair, 0)
    plsc.subcore_barrier()
    pltpu.sync_copy(acc_sh.at[pl.ds(s * RPT, RPT)],
                    out_hbm.at[c, pl.ds(s * RPT, RPT)])


@jax.jit
def _scatter(h, src, dst, zeros):
    mesh = plsc.VectorSubcoreMesh(core_axis_name="c", subcore_axis_name="s")
    f = pl.kernel(
        _scatter_body,
        out_type=jax.ShapeDtypeStruct((NC, N_PAD, D_FEAT), jnp.float32),
        mesh=mesh,
        scratch_types=[
            pltpu.VMEM_SHARED((N_PAD, D_FEAT), jnp.float32),
            pltpu.VMEM((E_PER_W,), jnp.int32),
            pltpu.VMEM((E_PER_W,), jnp.int32),
            pltpu.VMEM((NB, CHUNK, D_FEAT), jnp.float32),
            pltpu.SemaphoreType.DMA,
            pltpu.SemaphoreType.DMA,
            pltpu.SemaphoreType.DMA,
        ],
    )
    return f(h, src, dst, zeros)


def kernel(node_features, edge_index, weight):
    src = edge_index[0].astype(jnp.int32).reshape(NW, E_PER_W)
    dst = edge_index[1].astype(jnp.int32).reshape(NW, E_PER_W)
    h = _elu(node_features, weight)
    zeros = jnp.zeros((RPT, D_FEAT), jnp.float32)
    partials = _scatter(h, src, dst, zeros)
    return _sum_partials(partials)


# speedup vs baseline: 1.1220x; 1.1220x over previous
import functools

import jax
import jax.numpy as jnp
from jax import lax
from jax.experimental import pallas as pl
from jax.experimental.pallas import tpu as pltpu
from jax.experimental.pallas import tpu_sc as plsc

N_NODES = 10000
N_EDGES = 320000
D_FEAT = 128

NC = 2
NS = 16
NW = NC * NS
E_PER_W = N_EDGES // NW
CHUNK = 80
N_CHUNKS = E_PER_W // CHUNK
NB = 2
N_PAD = 10112
RPT = N_PAD // NS


def _elu_body(x_ref, w_ref, o_ref):
    xw = x_ref[...] * w_ref[...]
    o_ref[...] = jnp.where(xw > 0, xw, jnp.exp(xw) - 1.0)


@jax.jit
def _elu(x, w):
    return pl.pallas_call(
        _elu_body,
        out_shape=jax.ShapeDtypeStruct((N_NODES, D_FEAT), jnp.float32),
    )(x, w)


def _sum_body(p_ref, o_ref):
    o_ref[...] = p_ref[0, :N_NODES] + p_ref[1, :N_NODES]


@jax.jit
def _sum_partials(p):
    return pl.pallas_call(
        _sum_body,
        out_shape=jax.ShapeDtypeStruct((N_NODES, D_FEAT), jnp.float32),
    )(p)


def _scatter_body(h_hbm, src_hbm, dst_hbm, zeros_hbm, out_hbm,
                  acc_sh, srcb, dstb, rows, sem0, sem1, psem):
    c = lax.axis_index("c")
    s = lax.axis_index("s")
    wid = s * NC + c
    sems = [sem0, sem1]

    acc_slice = acc_sh.at[pl.ds(s * RPT, RPT)]
    pltpu.async_copy(src_hbm.at[wid], srcb, psem)
    pltpu.async_copy(dst_hbm.at[wid], dstb, psem)
    pltpu.async_copy(zeros_hbm, acc_slice, psem)
    pltpu.make_async_copy(src_hbm.at[wid], srcb, psem).wait()
    pltpu.make_async_copy(dst_hbm.at[wid], dstb, psem).wait()
    pltpu.make_async_copy(zeros_hbm, acc_slice, psem).wait()
    plsc.subcore_barrier()

    def idx_slice(ref, i):
        return ref.at[pl.ds(pl.multiple_of(i * CHUNK, 8), CHUNK)]

    for b in range(NB):
        pltpu.async_copy(h_hbm.at[idx_slice(srcb, b)], rows.at[b], sems[b])

    def pair(j, carry):
        for b in range(NB):
            i = j * NB + b

            @pl.when(i < N_CHUNKS)
            def _():
                pltpu.make_async_copy(h_hbm.at[idx_slice(srcb, i)],
                                      rows.at[b], sems[b]).wait()
                pltpu.sync_copy(rows.at[b], acc_sh.at[idx_slice(dstb, i)],
                                add=True)
                nxt = i + NB

                @pl.when(nxt < N_CHUNKS)
                def _():
                    pltpu.async_copy(h_hbm.at[idx_slice(srcb, nxt)],
                                     rows.at[b], sems[b])

        return carry

    lax.fori_loop(0, (N_CHUNKS + NB - 1) // NB, pair, 0)
    plsc.subcore_barrier()
    pltpu.sync_copy(acc_sh.at[pl.ds(s * RPT, RPT)],
                    out_hbm.at[c, pl.ds(s * RPT, RPT)])


@jax.jit
def _scatter(h, src, dst, zeros):
    mesh = plsc.VectorSubcoreMesh(core_axis_name="c", subcore_axis_name="s")
    f = pl.kernel(
        _scatter_body,
        out_type=jax.ShapeDtypeStruct((NC, N_PAD, D_FEAT), jnp.float32),
        mesh=mesh,
        scratch_types=[
            pltpu.VMEM_SHARED((N_PAD, D_FEAT), jnp.float32),
            pltpu.VMEM((E_PER_W,), jnp.int32),
            pltpu.VMEM((E_PER_W,), jnp.int32),
            pltpu.VMEM((NB, CHUNK, D_FEAT), jnp.float32),
            pltpu.SemaphoreType.DMA,
            pltpu.SemaphoreType.DMA,
            pltpu.SemaphoreType.DMA,
        ],
    )
    return f(h, src, dst, zeros)


def kernel(node_features, edge_index, weight):
    src = edge_index[0].astype(jnp.int32).reshape(NW, E_PER_W)
    dst = edge_index[1].astype(jnp.int32).reshape(NW, E_PER_W)
    h = _elu(node_features, weight)
    zeros = jnp.zeros((RPT, D_FEAT), jnp.float32)
    partials = _scatter(h, src, dst, zeros)
    return _sum_partials(partials)
